# baseline (speedup 1.0000x reference)
import functools

import jax
import jax.numpy as jnp
from jax.experimental import pallas as pl
from jax.experimental.pallas import tpu as pltpu

_GROUP = 512
_WIN = _GROUP + 128
_NEG = -3e38


def _scan_kernel(x_ref, bias_ref, m_ref, a_ref, *, ng, ngp, v, b):
    i = pl.program_id(0)

    @pl.when(i == 0)
    def _():
        m_ref[...] = jnp.full(m_ref.shape, _NEG, jnp.float32)
        a_ref[...] = jnp.zeros(a_ref.shape, jnp.float32)

    x = x_ref[...]
    bias = bias_ref[...]
    gcol = i * _GROUP + jax.lax.broadcasted_iota(jnp.int32, (b, _GROUP), 1)
    s = x + bias
    s = jnp.where(gcol == v - 1, s - 1000.0, s)
    s = jnp.where(gcol < v, s, _NEG)
    m = jnp.max(s, axis=1, keepdims=True)
    amx = jnp.min(jnp.where(s == m, gcol, 2 ** 30), axis=1, keepdims=True)
    giota = jax.lax.broadcasted_iota(jnp.int32, (b, ngp), 1)
    m_ref[...] = jnp.where(giota == i, m, m_ref[...])
    a_ref[...] = jnp.where(giota == i, amx.astype(jnp.float32), a_ref[...])


def _tourney_kernel(m_in, a_in, bias_ref, state_ref, seq_ref, bslp_ref,
                    x_hbm,
                    p_out, r_out, c_out, nstate_out, nseq_out, nbslp_out,
                    ms, as_, row, sem, *, k, ngp, vp, v, b, h, t):
    ms[...] = m_in[...]
    as_[...] = a_in[...]
    bias = bias_ref[...]
    riota = jax.lax.broadcasted_iota(jnp.int32, (b, ngp), 0)
    giota = jax.lax.broadcasted_iota(jnp.int32, (b, ngp), 1)
    flat = riota * ngp + giota
    oiota = jax.lax.broadcasted_iota(jnp.int32, (1, k), 1)
    piota = jax.lax.broadcasted_iota(jnp.int32, (b, 1), 0)
    pos = jax.lax.broadcasted_iota(jnp.int32, (1, _WIN), 1)

    def body(i, carry):
        vals, rows, cols, rlocs, picks = carry
        M = ms[...]
        m = jnp.max(M)
        idx = jnp.min(jnp.where(M == m, flat, 2 ** 30))
        r = idx // ngp
        g = idx % ngp
        a = jnp.sum(jnp.where(flat == idx, as_[...], 0.0))
        brow = jnp.sum(jnp.where(piota[:, 0:1] == r, bias, 0.0))
        rf = r.astype(jnp.float32)
        vals = jnp.where(oiota == i, m, vals)
        rows = jnp.where(oiota == i, rf, rows)
        cols = jnp.where(oiota == i, a, cols)
        rlocs = jnp.where(oiota == i, m - brow, rlocs)
        pick_id = rf * float(vp) + a
        picks = jnp.where(piota == i, pick_id, picks)
        start = jnp.minimum(g * _GROUP, v - _GROUP)
        start0 = pl.multiple_of((start // 128) * 128, 128)
        r0 = pl.multiple_of((r // 8) * 8, 8)
        cp = pltpu.make_async_copy(
            x_hbm.at[pl.ds(r0, 8), pl.ds(start0, _WIN)], row, sem)
        cp.start()
        cp.wait()
        gcol = start0 + jax.lax.broadcasted_iota(jnp.int32, (8, _WIN), 1)
        rsel = jax.lax.broadcasted_iota(jnp.int32, (8, _WIN), 0) == (r - r0)
        sall = jnp.where(rsel, row[...], _NEG)
        ok = (gcol >= g * _GROUP) & (gcol < (g + 1) * _GROUP) & (gcol < v)
        sall = jnp.where(ok, sall + brow, _NEG)
        sall = jnp.where(gcol == v - 1, sall - 1000.0, sall)
        srow = jnp.max(sall, axis=0, keepdims=True)
        gcol1 = start0 + pos
        ids = rf * float(vp) + gcol1.astype(jnp.float32)
        consumed = jnp.any(picks == ids, axis=0, keepdims=True)
        srow = jnp.where(consumed, _NEG, srow)
        m2 = jnp.max(srow)
        a2 = jnp.min(jnp.where(srow == m2, gcol1, 2 ** 30)).astype(jnp.float32)
        sel = flat == idx
        ms[...] = jnp.where(sel, m2, M)
        as_[...] = jnp.where(sel, a2, as_[...])
        return vals, rows, cols, rlocs, picks

    init = (jnp.full((1, k), _NEG, jnp.float32),
            jnp.zeros((1, k), jnp.float32),
            jnp.zeros((1, k), jnp.float32),
            jnp.zeros((1, k), jnp.float32),
            jnp.full((b, 1), -1.0, jnp.float32))
    vals, rows, cols, rlocs, _ = jax.lax.fori_loop(0, k, body, init)

    p_out[...] = vals
    r_out[...] = rlocs
    c_out[...] = cols.astype(jnp.int32)
    q = rows.astype(jnp.int32)
    nstate_out[...] = jnp.take_along_axis(
        state_ref[...], jnp.broadcast_to(q, (h, b)), axis=1,
        mode="promise_in_bounds")
    nseq_out[...] = jnp.take_along_axis(
        seq_ref[...], jnp.broadcast_to(q, (t, b)), axis=1,
        mode="promise_in_bounds")
    nbslp_out[...] = jnp.take_along_axis(
        bslp_ref[...], jnp.broadcast_to(q, (t, b)), axis=1,
        mode="promise_in_bounds")


def kernel(logprobs, beam_logprobs_sum, beam_seq_prev, beam_seq_logprobs_prev,
           state):
    b, v = logprobs.shape
    t = beam_seq_prev.shape[0]
    h = state.shape[0]
    k = b
    ng = pl.cdiv(v, _GROUP)
    ngp = max(256, ng)
    vp = ng * _GROUP

    bias = beam_logprobs_sum.reshape(b, 1).astype(jnp.float32)
    seq_f = beam_seq_prev.astype(jnp.float32)

    m_mat, a_mat = pl.pallas_call(
        functools.partial(_scan_kernel, ng=ng, ngp=ngp, v=v, b=b),
        grid=(ng,),
        in_specs=[
            pl.BlockSpec((b, _GROUP), lambda i: (0, i)),
            pl.BlockSpec((b, 1), lambda i: (0, 0)),
        ],
        out_specs=[
            pl.BlockSpec((b, ngp), lambda i: (0, 0)),
            pl.BlockSpec((b, ngp), lambda i: (0, 0)),
        ],
        out_shape=[
            jax.ShapeDtypeStruct((b, ngp), jnp.float32),
            jax.ShapeDtypeStruct((b, ngp), jnp.float32),
        ],
    )(logprobs, bias)

    top_p, r_loc, c_tok, nstate, nseq, nbslp = pl.pallas_call(
        functools.partial(_tourney_kernel, k=k, ngp=ngp, vp=vp, v=v, b=b,
                          h=h, t=t),
        in_specs=[
            pl.BlockSpec((b, ngp), lambda: (0, 0)),
            pl.BlockSpec((b, ngp), lambda: (0, 0)),
            pl.BlockSpec((b, 1), lambda: (0, 0)),
            pl.BlockSpec((h, b), lambda: (0, 0)),
            pl.BlockSpec((t, b), lambda: (0, 0)),
            pl.BlockSpec((t, b), lambda: (0, 0)),
            pl.BlockSpec(memory_space=pl.ANY),
        ],
        out_specs=[
            pl.BlockSpec((1, k), lambda: (0, 0)),
            pl.BlockSpec((1, k), lambda: (0, 0)),
            pl.BlockSpec((1, k), lambda: (0, 0)),
            pl.BlockSpec((h, b), lambda: (0, 0)),
            pl.BlockSpec((t, b), lambda: (0, 0)),
            pl.BlockSpec((t, b), lambda: (0, 0)),
        ],
        out_shape=[
            jax.ShapeDtypeStruct((1, k), jnp.float32),
            jax.ShapeDtypeStruct((1, k), jnp.float32),
            jax.ShapeDtypeStruct((1, k), jnp.int32),
            jax.ShapeDtypeStruct((h, b), jnp.float32),
            jax.ShapeDtypeStruct((t, b), jnp.float32),
            jax.ShapeDtypeStruct((t, b), jnp.float32),
        ],
        scratch_shapes=[
            pltpu.VMEM((b, ngp), jnp.float32),
            pltpu.VMEM((b, ngp), jnp.float32),
            pltpu.VMEM((8, _WIN), jnp.float32),
            pltpu.SemaphoreType.DMA,
        ],
    )(m_mat, a_mat, bias, state, seq_f, beam_seq_logprobs_prev, logprobs)

    c_row = c_tok.reshape(1, k).astype(beam_seq_prev.dtype)
    new_beam_seq = jnp.concatenate(
        [jnp.round(nseq).astype(beam_seq_prev.dtype), c_row], axis=0)
    new_beam_seq_logprobs = jnp.concatenate(
        [nbslp, r_loc.reshape(1, k)], axis=0)
    return (new_beam_seq, new_beam_seq_logprobs, top_p.reshape(k),
            nstate)


# speedup vs baseline: 1.2987x; 1.2987x over previous
import functools

import jax
import jax.numpy as jnp
from jax.experimental import pallas as pl
from jax.experimental.pallas import tpu as pltpu

_GROUP = 512
_WIN = _GROUP + 128
_NEG = -3e38


def _scan_kernel(x_ref, bias_ref, m_ref, a_ref, m2_ref, a2_ref,
                 *, ng, ngp, v, b):
    i = pl.program_id(0)

    @pl.when(i == 0)
    def _():
        m_ref[...] = jnp.full(m_ref.shape, _NEG, jnp.float32)
        a_ref[...] = jnp.zeros(a_ref.shape, jnp.float32)
        m2_ref[...] = jnp.full(m2_ref.shape, _NEG, jnp.float32)
        a2_ref[...] = jnp.zeros(a2_ref.shape, jnp.float32)

    x = x_ref[...]
    bias = bias_ref[...]
    gcol = i * _GROUP + jax.lax.broadcasted_iota(jnp.int32, (b, _GROUP), 1)
    s = x + bias
    s = jnp.where(gcol == v - 1, s - 1000.0, s)
    s = jnp.where(gcol < v, s, _NEG)
    m = jnp.max(s, axis=1, keepdims=True)
    amx = jnp.min(jnp.where(s == m, gcol, 2 ** 30), axis=1, keepdims=True)
    s_b = jnp.where(gcol == amx, _NEG, s)
    m2 = jnp.max(s_b, axis=1, keepdims=True)
    am2 = jnp.min(jnp.where(s_b == m2, gcol, 2 ** 30), axis=1, keepdims=True)
    giota = jax.lax.broadcasted_iota(jnp.int32, (b, ngp), 1)
    m_ref[...] = jnp.where(giota == i, m, m_ref[...])
    a_ref[...] = jnp.where(giota == i, amx.astype(jnp.float32), a_ref[...])
    m2_ref[...] = jnp.where(giota == i, m2, m2_ref[...])
    a2_ref[...] = jnp.where(giota == i, am2.astype(jnp.float32), a2_ref[...])


def _tourney_kernel(m_in, a_in, m2_in, a2_in, bias_ref, state_ref, seq_ref,
                    bslp_ref, x_hbm,
                    p_out, r_out, c_out, nstate_out, nseq_out, nbslp_out,
                    ms, as_, ms2, as2, row, sem, *, k, ngp, vp, v, b, h, t):
    ms[...] = m_in[...]
    as_[...] = a_in[...]
    ms2[...] = m2_in[...]
    as2[...] = a2_in[...]
    bias = bias_ref[...]
    riota = jax.lax.broadcasted_iota(jnp.int32, (b, ngp), 0)
    giota = jax.lax.broadcasted_iota(jnp.int32, (b, ngp), 1)
    flat = riota * ngp + giota
    oiota = jax.lax.broadcasted_iota(jnp.int32, (1, k), 1)
    piota = jax.lax.broadcasted_iota(jnp.int32, (b, 1), 0)
    pos = jax.lax.broadcasted_iota(jnp.int32, (1, _WIN), 1)

    def body(i, carry):
        vals, rows, cols, rlocs, picks = carry
        M = ms[...]
        m = jnp.max(M)
        idx = jnp.min(jnp.where(M == m, flat, 2 ** 30))
        r = idx // ngp
        g = idx % ngp
        a = jnp.sum(jnp.where(flat == idx, as_[...], 0.0))
        brow = jnp.sum(jnp.where(piota[:, 0:1] == r, bias, 0.0))
        rf = r.astype(jnp.float32)
        vals = jnp.where(oiota == i, m, vals)
        rows = jnp.where(oiota == i, rf, rows)
        cols = jnp.where(oiota == i, a, cols)
        rlocs = jnp.where(oiota == i, m - brow, rlocs)
        pick_id = rf * float(vp) + a
        picks = jnp.where(piota == i, pick_id, picks)
        sel = flat == idx
        bk = jnp.sum(jnp.where(sel, ms2[...], 0.0))
        ab = jnp.sum(jnp.where(sel, as2[...], 0.0))
        ms[...] = jnp.where(sel, bk, M)
        as_[...] = jnp.where(sel, ab, as_[...])
        ms2[...] = jnp.where(sel, _NEG, ms2[...])

        @pl.when(bk < -1e37)
        def _():
            start = jnp.minimum(g * _GROUP, v - _GROUP)
            start0 = pl.multiple_of((start // 128) * 128, 128)
            r0 = pl.multiple_of((r // 8) * 8, 8)
            cp = pltpu.make_async_copy(
                x_hbm.at[pl.ds(r0, 8), pl.ds(start0, _WIN)], row, sem)
            cp.start()
            cp.wait()
            gcol = start0 + jax.lax.broadcasted_iota(jnp.int32, (8, _WIN), 1)
            rsel = (jax.lax.broadcasted_iota(jnp.int32, (8, _WIN), 0)
                    == (r - r0))
            sall = jnp.where(rsel, row[...], _NEG)
            ok = (gcol >= g * _GROUP) & (gcol < (g + 1) * _GROUP) & (gcol < v)
            sall = jnp.where(ok, sall + brow, _NEG)
            sall = jnp.where(gcol == v - 1, sall - 1000.0, sall)
            srow = jnp.max(sall, axis=0, keepdims=True)
            gcol1 = start0 + pos
            ids = rf * float(vp) + gcol1.astype(jnp.float32)
            consumed = jnp.any(picks == ids, axis=0, keepdims=True)
            srow = jnp.where(consumed, _NEG, srow)
            m2 = jnp.max(srow)
            a2 = jnp.min(
                jnp.where(srow == m2, gcol1, 2 ** 30)).astype(jnp.float32)
            srow_b = jnp.where(gcol1 == a2.astype(jnp.int32), _NEG, srow)
            m2b = jnp.max(srow_b)
            a2b = jnp.min(
                jnp.where(srow_b == m2b, gcol1, 2 ** 30)).astype(jnp.float32)
            ms[...] = jnp.where(sel, m2, ms[...])
            as_[...] = jnp.where(sel, a2, as_[...])
            ms2[...] = jnp.where(sel, m2b, ms2[...])
            as2[...] = jnp.where(sel, a2b, as2[...])

        return vals, rows, cols, rlocs, picks

    init = (jnp.full((1, k), _NEG, jnp.float32),
            jnp.zeros((1, k), jnp.float32),
            jnp.zeros((1, k), jnp.float32),
            jnp.zeros((1, k), jnp.float32),
            jnp.full((b, 1), -1.0, jnp.float32))
    vals, rows, cols, rlocs, _ = jax.lax.fori_loop(0, k, body, init)

    p_out[...] = vals
    r_out[...] = rlocs
    c_out[...] = cols.astype(jnp.int32)
    q = rows.astype(jnp.int32)
    nstate_out[...] = jnp.take_along_axis(
        state_ref[...], jnp.broadcast_to(q, (h, b)), axis=1,
        mode="promise_in_bounds")
    nseq_out[...] = jnp.take_along_axis(
        seq_ref[...], jnp.broadcast_to(q, (t, b)), axis=1,
        mode="promise_in_bounds")
    nbslp_out[...] = jnp.take_along_axis(
        bslp_ref[...], jnp.broadcast_to(q, (t, b)), axis=1,
        mode="promise_in_bounds")


def kernel(logprobs, beam_logprobs_sum, beam_seq_prev, beam_seq_logprobs_prev,
           state):
    b, v = logprobs.shape
    t = beam_seq_prev.shape[0]
    h = state.shape[0]
    k = b
    ng = pl.cdiv(v, _GROUP)
    ngp = max(256, ng)
    vp = ng * _GROUP

    bias = beam_logprobs_sum.reshape(b, 1).astype(jnp.float32)
    seq_f = beam_seq_prev.astype(jnp.float32)

    m_mat, a_mat, m2_mat, a2_mat = pl.pallas_call(
        functools.partial(_scan_kernel, ng=ng, ngp=ngp, v=v, b=b),
        grid=(ng,),
        in_specs=[
            pl.BlockSpec((b, _GROUP), lambda i: (0, i)),
            pl.BlockSpec((b, 1), lambda i: (0, 0)),
        ],
        out_specs=[pl.BlockSpec((b, ngp), lambda i: (0, 0))] * 4,
        out_shape=[jax.ShapeDtypeStruct((b, ngp), jnp.float32)] * 4,
    )(logprobs, bias)

    top_p, r_loc, c_tok, nstate, nseq, nbslp = pl.pallas_call(
        functools.partial(_tourney_kernel, k=k, ngp=ngp, vp=vp, v=v, b=b,
                          h=h, t=t),
        in_specs=[
            pl.BlockSpec((b, ngp), lambda: (0, 0)),
            pl.BlockSpec((b, ngp), lambda: (0, 0)),
            pl.BlockSpec((b, ngp), lambda: (0, 0)),
            pl.BlockSpec((b, ngp), lambda: (0, 0)),
            pl.BlockSpec((b, 1), lambda: (0, 0)),
            pl.BlockSpec((h, b), lambda: (0, 0)),
            pl.BlockSpec((t, b), lambda: (0, 0)),
            pl.BlockSpec((t, b), lambda: (0, 0)),
            pl.BlockSpec(memory_space=pl.ANY),
        ],
        out_specs=[
            pl.BlockSpec((1, k), lambda: (0, 0)),
            pl.BlockSpec((1, k), lambda: (0, 0)),
            pl.BlockSpec((1, k), lambda: (0, 0)),
            pl.BlockSpec((h, b), lambda: (0, 0)),
            pl.BlockSpec((t, b), lambda: (0, 0)),
            pl.BlockSpec((t, b), lambda: (0, 0)),
        ],
        out_shape=[
            jax.ShapeDtypeStruct((1, k), jnp.float32),
            jax.ShapeDtypeStruct((1, k), jnp.float32),
            jax.ShapeDtypeStruct((1, k), jnp.int32),
            jax.ShapeDtypeStruct((h, b), jnp.float32),
            jax.ShapeDtypeStruct((t, b), jnp.float32),
            jax.ShapeDtypeStruct((t, b), jnp.float32),
        ],
        scratch_shapes=[
            pltpu.VMEM((b, ngp), jnp.float32),
            pltpu.VMEM((b, ngp), jnp.float32),
            pltpu.VMEM((b, ngp), jnp.float32),
            pltpu.VMEM((b, ngp), jnp.float32),
            pltpu.VMEM((8, _WIN), jnp.float32),
            pltpu.SemaphoreType.DMA,
        ],
    )(m_mat, a_mat, m2_mat, a2_mat, bias, state, seq_f,
      beam_seq_logprobs_prev, logprobs)

    c_row = c_tok.reshape(1, k).astype(beam_seq_prev.dtype)
    new_beam_seq = jnp.concatenate(
        [jnp.round(nseq).astype(beam_seq_prev.dtype), c_row], axis=0)
    new_beam_seq_logprobs = jnp.concatenate(
        [nbslp, r_loc.reshape(1, k)], axis=0)
    return (new_beam_seq, new_beam_seq_logprobs, top_p.reshape(k),
            nstate)
